# baseline (speedup 1.0000x reference)
import functools
import jax
import jax.numpy as jnp
from jax.experimental import pallas as pl
from jax.experimental.pallas import tpu as pltpu

N = 16384
L = 1024
D = 128
F = 32
B = 16

TILE = 1024
NT = N // TILE
NEG = -1e30


def _fused_kernel(x_ref, idx_ref, W1_ref, b1_ref, Wt_ref, bt_ref, Ws_ref,
                  bs_ref, WaB_ref, baB_ref, Wp_ref, bp_ref,
                  M_out, proj_out,
                  m_scr, s_scr, S_scr):
    i = pl.program_id(0)

    @pl.when(i == 0)
    def _init():
        m_scr[...] = jnp.full((1, B), NEG, jnp.float32)
        s_scr[...] = jnp.zeros((1, B), jnp.float32)
        S_scr[...] = jnp.zeros((B, D), jnp.float32)

    x = x_ref[...].astype(jnp.bfloat16)
    H = jax.lax.dot_general(x, W1_ref[...], (((1,), (1,)), ((), ())),
                            preferred_element_type=jnp.float32)
    H = jnp.maximum(H + b1_ref[...], 0.0)

    At = jnp.tanh(jax.lax.dot_general(H, Wt_ref[...], (((1,), (1,)), ((), ())),
                                      preferred_element_type=jnp.float32)
                  + bt_ref[...])
    As = jax.nn.sigmoid(
        jax.lax.dot_general(H, Ws_ref[...], (((1,), (1,)), ((), ())),
                            preferred_element_type=jnp.float32)
        + bs_ref[...])
    aB = (jax.lax.dot_general(At * As, WaB_ref[...], (((1,), (0,)), ((), ())),
                              preferred_element_type=jnp.float32)
          + baB_ref[...])

    idx = idx_ref[0]
    seg = jax.lax.broadcasted_iota(jnp.int32, (TILE, B), 1)
    onehot = (idx.reshape(TILE, 1) == seg)

    a_b = jnp.where(onehot, aB, NEG)
    tile_max = jnp.max(a_b, axis=0, keepdims=True)

    m_old = m_scr[...]
    m_new = jnp.maximum(m_old, tile_max)
    alpha = jnp.exp(m_old - m_new)

    z = jnp.where(onehot, a_b - m_new, NEG)
    e = jnp.exp(z)

    m_scr[...] = m_new
    s_scr[...] = s_scr[...] * alpha + jnp.sum(e, axis=0, keepdims=True)
    S_scr[...] = (S_scr[...] * alpha.reshape(B, 1)
                  + jax.lax.dot_general(e, H, (((0,), (0,)), ((), ())),
                                        preferred_element_type=jnp.float32))

    @pl.when(i == NT - 1)
    def _finish():
        s = s_scr[...].reshape(B, 1)
        M = S_scr[...] / jnp.where(s > 0.0, s, 1.0)
        M_out[...] = M
        proj = (jax.lax.dot_general(M, Wp_ref[...], (((1,), (1,)), ((), ())),
                                    preferred_element_type=jnp.float32)
                + bp_ref[...])
        nrm = jnp.sqrt(jnp.sum(proj * proj, axis=1, keepdims=True))
        proj_out[...] = proj / jnp.maximum(nrm, 1e-12)


@jax.jit
def kernel(x, idxs, W1, b1, Wt, bt, Ws, bs, Wa, ba, Wp, bp):
    idx3 = idxs.astype(jnp.int32).reshape(NT, 1, TILE)
    W1c = W1.astype(jnp.bfloat16)
    WaB = jnp.broadcast_to(Wa.reshape(F, 1), (F, B))
    baB = jnp.broadcast_to(ba.reshape(1, 1), (1, B))
    b1r = b1.reshape(1, D)
    btr = bt.reshape(1, F)
    bsr = bs.reshape(1, F)
    bpr = bp.reshape(1, F)
    grid = (NT,)
    out = pl.pallas_call(
        _fused_kernel,
        grid=grid,
        in_specs=[
            pl.BlockSpec((TILE, L), lambda i: (i, 0)),
            pl.BlockSpec((1, 1, TILE), lambda i: (i, 0, 0)),
            pl.BlockSpec((D, L), lambda i: (0, 0)),
            pl.BlockSpec((1, D), lambda i: (0, 0)),
            pl.BlockSpec((F, D), lambda i: (0, 0)),
            pl.BlockSpec((1, F), lambda i: (0, 0)),
            pl.BlockSpec((F, D), lambda i: (0, 0)),
            pl.BlockSpec((1, F), lambda i: (0, 0)),
            pl.BlockSpec((F, B), lambda i: (0, 0)),
            pl.BlockSpec((1, B), lambda i: (0, 0)),
            pl.BlockSpec((F, D), lambda i: (0, 0)),
            pl.BlockSpec((1, F), lambda i: (0, 0)),
        ],
        out_specs=[
            pl.BlockSpec((B, D), lambda i: (0, 0)),
            pl.BlockSpec((B, F), lambda i: (0, 0)),
        ],
        out_shape=[
            jax.ShapeDtypeStruct((B, D), jnp.float32),
            jax.ShapeDtypeStruct((B, F), jnp.float32),
        ],
        scratch_shapes=[
            pltpu.VMEM((1, B), jnp.float32),
            pltpu.VMEM((1, B), jnp.float32),
            pltpu.VMEM((B, D), jnp.float32),
        ],
    )(x, idx3, W1c, b1r, Wt, btr, Ws, bsr, WaB, baB, Wp, bpr)
    M, proj = out
    return (M, proj)


# speedup vs baseline: 1.1645x; 1.1645x over previous
import functools
import jax
import jax.numpy as jnp
from jax.experimental import pallas as pl
from jax.experimental.pallas import tpu as pltpu

N = 16384
L = 1024
D = 128
F = 32
B = 16

TILE = 2048
NT = N // TILE
NEG = -1e30


def _fused_kernel(x_ref, idx_ref, W1_ref, b1_ref, Wt_ref, bt_ref, Ws_ref,
                  bs_ref, WaB_ref, baB_ref, Wp_ref, bp_ref,
                  M_out, proj_out,
                  m_scr, s_scr, S_scr):
    i = pl.program_id(0)

    @pl.when(i == 0)
    def _init():
        m_scr[...] = jnp.full((1, B), NEG, jnp.float32)
        s_scr[...] = jnp.zeros((1, B), jnp.float32)
        S_scr[...] = jnp.zeros((B, D), jnp.float32)

    x = x_ref[...]
    H = jax.lax.dot_general(x, W1_ref[...], (((1,), (1,)), ((), ())),
                            preferred_element_type=jnp.float32)
    H = jnp.maximum(H + b1_ref[...], 0.0)

    At = jnp.tanh(jax.lax.dot_general(H, Wt_ref[...], (((1,), (1,)), ((), ())),
                                      preferred_element_type=jnp.float32)
                  + bt_ref[...])
    As = jax.nn.sigmoid(
        jax.lax.dot_general(H, Ws_ref[...], (((1,), (1,)), ((), ())),
                            preferred_element_type=jnp.float32)
        + bs_ref[...])
    aB = (jax.lax.dot_general(At * As, WaB_ref[...], (((1,), (0,)), ((), ())),
                              preferred_element_type=jnp.float32)
          + baB_ref[...])

    idx = idx_ref[0]
    seg = jax.lax.broadcasted_iota(jnp.int32, (TILE, B), 1)
    onehot = (idx.reshape(TILE, 1) == seg)

    a_b = jnp.where(onehot, aB, NEG)
    tile_max = jnp.max(a_b, axis=0, keepdims=True)

    m_old = m_scr[...]
    m_new = jnp.maximum(m_old, tile_max)
    alpha = jnp.exp(m_old - m_new)

    z = jnp.where(onehot, a_b - m_new, NEG)
    e = jnp.exp(z)

    m_scr[...] = m_new
    s_scr[...] = s_scr[...] * alpha + jnp.sum(e, axis=0, keepdims=True)
    S_scr[...] = (S_scr[...] * alpha.reshape(B, 1)
                  + jax.lax.dot_general(e, H, (((0,), (0,)), ((), ())),
                                        preferred_element_type=jnp.float32))

    @pl.when(i == NT - 1)
    def _finish():
        s = s_scr[...].reshape(B, 1)
        M = S_scr[...] / jnp.where(s > 0.0, s, 1.0)
        M_out[...] = M
        proj = (jax.lax.dot_general(M, Wp_ref[...], (((1,), (1,)), ((), ())),
                                    preferred_element_type=jnp.float32)
                + bp_ref[...])
        nrm = jnp.sqrt(jnp.sum(proj * proj, axis=1, keepdims=True))
        proj_out[...] = proj / jnp.maximum(nrm, 1e-12)


@jax.jit
def kernel(x, idxs, W1, b1, Wt, bt, Ws, bs, Wa, ba, Wp, bp):
    idx3 = idxs.astype(jnp.int32).reshape(NT, 1, TILE)
    W1c = W1
    WaB = jnp.broadcast_to(Wa.reshape(F, 1), (F, B))
    baB = jnp.broadcast_to(ba.reshape(1, 1), (1, B))
    b1r = b1.reshape(1, D)
    btr = bt.reshape(1, F)
    bsr = bs.reshape(1, F)
    bpr = bp.reshape(1, F)
    grid = (NT,)
    out = pl.pallas_call(
        _fused_kernel,
        grid=grid,
        in_specs=[
            pl.BlockSpec((TILE, L), lambda i: (i, 0)),
            pl.BlockSpec((1, 1, TILE), lambda i: (i, 0, 0)),
            pl.BlockSpec((D, L), lambda i: (0, 0)),
            pl.BlockSpec((1, D), lambda i: (0, 0)),
            pl.BlockSpec((F, D), lambda i: (0, 0)),
            pl.BlockSpec((1, F), lambda i: (0, 0)),
            pl.BlockSpec((F, D), lambda i: (0, 0)),
            pl.BlockSpec((1, F), lambda i: (0, 0)),
            pl.BlockSpec((F, B), lambda i: (0, 0)),
            pl.BlockSpec((1, B), lambda i: (0, 0)),
            pl.BlockSpec((F, D), lambda i: (0, 0)),
            pl.BlockSpec((1, F), lambda i: (0, 0)),
        ],
        out_specs=[
            pl.BlockSpec((B, D), lambda i: (0, 0)),
            pl.BlockSpec((B, F), lambda i: (0, 0)),
        ],
        out_shape=[
            jax.ShapeDtypeStruct((B, D), jnp.float32),
            jax.ShapeDtypeStruct((B, F), jnp.float32),
        ],
        scratch_shapes=[
            pltpu.VMEM((1, B), jnp.float32),
            pltpu.VMEM((1, B), jnp.float32),
            pltpu.VMEM((B, D), jnp.float32),
        ],
    )(x, idx3, W1c, b1r, Wt, btr, Ws, bsr, WaB, baB, Wp, bpr)
    M, proj = out
    return (M, proj)


# speedup vs baseline: 1.1913x; 1.0230x over previous
import functools
import jax
import jax.numpy as jnp
from jax.experimental import pallas as pl
from jax.experimental.pallas import tpu as pltpu

N = 16384
L = 1024
D = 128
F = 32
B = 16

TILE = 4096
NT = N // TILE
NEG = -1e30


def _fused_kernel(x_ref, idx_ref, W1_ref, b1_ref, Wt_ref, bt_ref, Ws_ref,
                  bs_ref, WaB_ref, baB_ref, Wp_ref, bp_ref,
                  M_out, proj_out,
                  m_scr, s_scr, S_scr):
    i = pl.program_id(0)

    @pl.when(i == 0)
    def _init():
        m_scr[...] = jnp.full((1, B), NEG, jnp.float32)
        s_scr[...] = jnp.zeros((1, B), jnp.float32)
        S_scr[...] = jnp.zeros((B, D), jnp.float32)

    x = x_ref[...]
    H = jax.lax.dot_general(x, W1_ref[...], (((1,), (1,)), ((), ())),
                            preferred_element_type=jnp.float32)
    H = jnp.maximum(H + b1_ref[...], 0.0)

    At = jnp.tanh(jax.lax.dot_general(H, Wt_ref[...], (((1,), (1,)), ((), ())),
                                      preferred_element_type=jnp.float32)
                  + bt_ref[...])
    As = jax.nn.sigmoid(
        jax.lax.dot_general(H, Ws_ref[...], (((1,), (1,)), ((), ())),
                            preferred_element_type=jnp.float32)
        + bs_ref[...])
    aB = (jax.lax.dot_general(At * As, WaB_ref[...], (((1,), (0,)), ((), ())),
                              preferred_element_type=jnp.float32)
          + baB_ref[...])

    idx = idx_ref[0]
    seg = jax.lax.broadcasted_iota(jnp.int32, (TILE, B), 1)
    onehot = (idx.reshape(TILE, 1) == seg)

    a_b = jnp.where(onehot, aB, NEG)
    tile_max = jnp.max(a_b, axis=0, keepdims=True)

    m_old = m_scr[...]
    m_new = jnp.maximum(m_old, tile_max)
    alpha = jnp.exp(m_old - m_new)

    z = jnp.where(onehot, a_b - m_new, NEG)
    e = jnp.exp(z)

    m_scr[...] = m_new
    s_scr[...] = s_scr[...] * alpha + jnp.sum(e, axis=0, keepdims=True)
    S_scr[...] = (S_scr[...] * alpha.reshape(B, 1)
                  + jax.lax.dot_general(e, H, (((0,), (0,)), ((), ())),
                                        preferred_element_type=jnp.float32))

    @pl.when(i == NT - 1)
    def _finish():
        s = s_scr[...].reshape(B, 1)
        M = S_scr[...] / jnp.where(s > 0.0, s, 1.0)
        M_out[...] = M
        proj = (jax.lax.dot_general(M, Wp_ref[...], (((1,), (1,)), ((), ())),
                                    preferred_element_type=jnp.float32)
                + bp_ref[...])
        nrm = jnp.sqrt(jnp.sum(proj * proj, axis=1, keepdims=True))
        proj_out[...] = proj / jnp.maximum(nrm, 1e-12)


@jax.jit
def kernel(x, idxs, W1, b1, Wt, bt, Ws, bs, Wa, ba, Wp, bp):
    idx3 = idxs.astype(jnp.int32).reshape(NT, 1, TILE)
    W1c = W1
    WaB = jnp.broadcast_to(Wa.reshape(F, 1), (F, B))
    baB = jnp.broadcast_to(ba.reshape(1, 1), (1, B))
    b1r = b1.reshape(1, D)
    btr = bt.reshape(1, F)
    bsr = bs.reshape(1, F)
    bpr = bp.reshape(1, F)
    grid = (NT,)
    out = pl.pallas_call(
        _fused_kernel,
        grid=grid,
        in_specs=[
            pl.BlockSpec((TILE, L), lambda i: (i, 0)),
            pl.BlockSpec((1, 1, TILE), lambda i: (i, 0, 0)),
            pl.BlockSpec((D, L), lambda i: (0, 0)),
            pl.BlockSpec((1, D), lambda i: (0, 0)),
            pl.BlockSpec((F, D), lambda i: (0, 0)),
            pl.BlockSpec((1, F), lambda i: (0, 0)),
            pl.BlockSpec((F, D), lambda i: (0, 0)),
            pl.BlockSpec((1, F), lambda i: (0, 0)),
            pl.BlockSpec((F, B), lambda i: (0, 0)),
            pl.BlockSpec((1, B), lambda i: (0, 0)),
            pl.BlockSpec((F, D), lambda i: (0, 0)),
            pl.BlockSpec((1, F), lambda i: (0, 0)),
        ],
        out_specs=[
            pl.BlockSpec((B, D), lambda i: (0, 0)),
            pl.BlockSpec((B, F), lambda i: (0, 0)),
        ],
        out_shape=[
            jax.ShapeDtypeStruct((B, D), jnp.float32),
            jax.ShapeDtypeStruct((B, F), jnp.float32),
        ],
        scratch_shapes=[
            pltpu.VMEM((1, B), jnp.float32),
            pltpu.VMEM((1, B), jnp.float32),
            pltpu.VMEM((B, D), jnp.float32),
        ],
    )(x, idx3, W1c, b1r, Wt, btr, Ws, bsr, WaB, baB, Wp, bpr)
    M, proj = out
    return (M, proj)


# speedup vs baseline: 1.2007x; 1.0079x over previous
import functools
import jax
import jax.numpy as jnp
from jax.experimental import pallas as pl
from jax.experimental.pallas import tpu as pltpu

N = 16384
L = 1024
D = 128
F = 32
B = 16

TILE = 4096
NT = N // TILE
NEG = -1e30


def _fused_kernel(xa_ref, xb_ref, idx_ref, W1_ref, b1_ref, Wt_ref, bt_ref,
                  Ws_ref, bs_ref, WaB_ref, baB_ref, Wp_ref, bp_ref,
                  M_out, proj_out,
                  m_scr, s_scr, S_scr):
    i = pl.program_id(0)

    @pl.when(i == 0)
    def _init():
        m_scr[...] = jnp.full((1, B), NEG, jnp.float32)
        s_scr[...] = jnp.zeros((1, B), jnp.float32)
        S_scr[...] = jnp.zeros((B, D), jnp.float32)

    W1 = W1_ref[...]
    Ha = jax.lax.dot_general(xa_ref[...], W1, (((1,), (1,)), ((), ())),
                             preferred_element_type=jnp.float32)
    Hb = jax.lax.dot_general(xb_ref[...], W1, (((1,), (1,)), ((), ())),
                             preferred_element_type=jnp.float32)
    H = jnp.concatenate([Ha, Hb], axis=0)
    H = jnp.maximum(H + b1_ref[...], 0.0)

    At = jnp.tanh(jax.lax.dot_general(H, Wt_ref[...], (((1,), (1,)), ((), ())),
                                      preferred_element_type=jnp.float32)
                  + bt_ref[...])
    As = jax.nn.sigmoid(
        jax.lax.dot_general(H, Ws_ref[...], (((1,), (1,)), ((), ())),
                            preferred_element_type=jnp.float32)
        + bs_ref[...])
    aB = (jax.lax.dot_general(At * As, WaB_ref[...], (((1,), (0,)), ((), ())),
                              preferred_element_type=jnp.float32)
          + baB_ref[...])

    idx = idx_ref[0]
    seg = jax.lax.broadcasted_iota(jnp.int32, (TILE, B), 1)
    onehot = (idx.reshape(TILE, 1) == seg)

    a_b = jnp.where(onehot, aB, NEG)
    tile_max = jnp.max(a_b, axis=0, keepdims=True)

    m_old = m_scr[...]
    m_new = jnp.maximum(m_old, tile_max)
    alpha = jnp.exp(m_old - m_new)

    z = jnp.where(onehot, a_b - m_new, NEG)
    e = jnp.exp(z)

    m_scr[...] = m_new
    s_scr[...] = s_scr[...] * alpha + jnp.sum(e, axis=0, keepdims=True)
    S_scr[...] = (S_scr[...] * alpha.reshape(B, 1)
                  + jax.lax.dot_general(e, H, (((0,), (0,)), ((), ())),
                                        preferred_element_type=jnp.float32))

    @pl.when(i == NT - 1)
    def _finish():
        s = s_scr[...].reshape(B, 1)
        M = S_scr[...] / jnp.where(s > 0.0, s, 1.0)
        M_out[...] = M
        proj = (jax.lax.dot_general(M, Wp_ref[...], (((1,), (1,)), ((), ())),
                                    preferred_element_type=jnp.float32)
                + bp_ref[...])
        nrm = jnp.sqrt(jnp.sum(proj * proj, axis=1, keepdims=True))
        proj_out[...] = proj / jnp.maximum(nrm, 1e-12)


@jax.jit
def kernel(x, idxs, W1, b1, Wt, bt, Ws, bs, Wa, ba, Wp, bp):
    idx3 = idxs.astype(jnp.int32).reshape(NT, 1, TILE)
    W1c = W1
    WaB = jnp.broadcast_to(Wa.reshape(F, 1), (F, B))
    baB = jnp.broadcast_to(ba.reshape(1, 1), (1, B))
    b1r = b1.reshape(1, D)
    btr = bt.reshape(1, F)
    bsr = bs.reshape(1, F)
    bpr = bp.reshape(1, F)
    grid = (NT,)
    out = pl.pallas_call(
        _fused_kernel,
        grid=grid,
        in_specs=[
            pl.BlockSpec((TILE // 2, L), lambda i: (2 * i, 0)),
            pl.BlockSpec((TILE // 2, L), lambda i: (2 * i + 1, 0)),
            pl.BlockSpec((1, 1, TILE), lambda i: (i, 0, 0)),
            pl.BlockSpec((D, L), lambda i: (0, 0)),
            pl.BlockSpec((1, D), lambda i: (0, 0)),
            pl.BlockSpec((F, D), lambda i: (0, 0)),
            pl.BlockSpec((1, F), lambda i: (0, 0)),
            pl.BlockSpec((F, D), lambda i: (0, 0)),
            pl.BlockSpec((1, F), lambda i: (0, 0)),
            pl.BlockSpec((F, B), lambda i: (0, 0)),
            pl.BlockSpec((1, B), lambda i: (0, 0)),
            pl.BlockSpec((F, D), lambda i: (0, 0)),
            pl.BlockSpec((1, F), lambda i: (0, 0)),
        ],
        out_specs=[
            pl.BlockSpec((B, D), lambda i: (0, 0)),
            pl.BlockSpec((B, F), lambda i: (0, 0)),
        ],
        out_shape=[
            jax.ShapeDtypeStruct((B, D), jnp.float32),
            jax.ShapeDtypeStruct((B, F), jnp.float32),
        ],
        scratch_shapes=[
            pltpu.VMEM((1, B), jnp.float32),
            pltpu.VMEM((1, B), jnp.float32),
            pltpu.VMEM((B, D), jnp.float32),
        ],
    )(x, x, idx3, W1c, b1r, Wt, btr, Ws, bsr, WaB, baB, Wp, bpr)
    M, proj = out
    return (M, proj)
